# baseline (speedup 1.0000x reference)
import functools

import jax
import jax.numpy as jnp
from jax import lax
from jax.experimental import pallas as pl
from jax.experimental.pallas import tpu as pltpu
from jax.experimental.pallas import tpu_sc as plsc

N_TOKENS = 100000
N_EMBD = 128

_B0, _B1 = 16384, 50
_TPU_ = 2
_NBUF = 4


def _make_gather():
    info = plsc.get_sparse_core_info()
    nw = info.num_cores * info.num_subcores
    toks_per_w = _B0 // nw
    units_per_w = toks_per_w // _TPU_
    ngrp = units_per_w // _NBUF
    mesh = plsc.VectorSubcoreMesh(core_axis_name="c", subcore_axis_name="s")

    @functools.partial(
        pl.kernel,
        out_type=jax.ShapeDtypeStruct((_B0, _B1, N_EMBD), jnp.float32),
        mesh=mesh,
        compiler_params=pltpu.CompilerParams(use_tc_tiling_on_sc=True,
                                             needs_layout_passes=True),
        scratch_types=(
            [pltpu.VMEM((toks_per_w, _B1), jnp.int32),
             pltpu.VMEM((_NBUF, _TPU_, _B1, N_EMBD), jnp.float32)]
            + [pltpu.SemaphoreType.DMA] * (2 * _NBUF)
        ),
    )
    def k(idx_hbm, table_hbm, out_hbm, idx_v, rows_v, *sems):
        gsem, wsem = sems[:_NBUF], sems[_NBUF:]
        wid = lax.axis_index("s") * info.num_cores + lax.axis_index("c")
        t_base = wid * toks_per_w
        pltpu.sync_copy(idx_hbm.at[pl.ds(t_base, toks_per_w)], idx_v)

        def g_issue(u, b):
            for t in range(_TPU_):
                pltpu.async_copy(table_hbm.at[idx_v.at[u * _TPU_ + t]],
                                 rows_v.at[b, t], gsem[b])

        def g_wait(u, b):
            for t in range(_TPU_):
                pltpu.make_async_copy(table_hbm.at[idx_v.at[u * _TPU_ + t]],
                                      rows_v.at[b, t], gsem[b]).wait()

        def w_issue(u, b):
            pltpu.async_copy(rows_v.at[b],
                             out_hbm.at[pl.ds(t_base + u * _TPU_, _TPU_)],
                             wsem[b])

        def w_wait(u, b):
            pltpu.make_async_copy(rows_v.at[b],
                                  out_hbm.at[pl.ds(t_base + u * _TPU_, _TPU_)],
                                  wsem[b]).wait()

        for b in range(_NBUF):
            g_issue(b, b)

        g_wait(0, 0)
        w_issue(0, 0)
        for i in range(1, _NBUF):
            g_wait(i, i)
            w_issue(i, i)
            w_wait(i - 1, i - 1)
            g_issue(i + _NBUF - 1, i - 1)

        def body(g, carry):
            for i in range(_NBUF):
                u = _NBUF * g + i
                g_wait(u, i)
                w_issue(u, i)
                bb = (i + _NBUF - 1) % _NBUF
                w_wait(u - 1, bb)
                g_issue(u + _NBUF - 1, bb)
            return carry

        lax.fori_loop(1, ngrp - 1, body, 0)

        u0 = _NBUF * (ngrp - 1)
        g_wait(u0, 0)
        w_issue(u0, 0)
        w_wait(u0 - 1, _NBUF - 1)
        g_issue(u0 + _NBUF - 1, _NBUF - 1)
        for i in range(1, _NBUF):
            g_wait(u0 + i, i)
            w_issue(u0 + i, i)

        for b in range(_NBUF):
            w_wait(u0 + b, b)

    return k


_gather = _make_gather()


def kernel(x, table):
    return _gather(x.astype(jnp.int32), table)


# speedup vs baseline: 1.9487x; 1.9487x over previous
import functools

import jax
import jax.numpy as jnp
from jax import lax
from jax.experimental import pallas as pl
from jax.experimental.pallas import tpu as pltpu
from jax.experimental.pallas import tpu_sc as plsc

N_TOKENS = 100000
N_EMBD = 128

_B0, _B1 = 16384, 50
_C = 128
_NBUF = 4


def _make_gather():
    info = plsc.get_sparse_core_info()
    nw = info.num_cores * info.num_subcores
    toks_per_w = _B0 // nw
    nsub = toks_per_w // _C
    nchunk = _B1 * nsub
    ngrp = nchunk // _NBUF
    mesh = plsc.VectorSubcoreMesh(core_axis_name="c", subcore_axis_name="s")

    @functools.partial(
        pl.kernel,
        out_type=jax.ShapeDtypeStruct((_B1, _B0, N_EMBD), jnp.float32),
        mesh=mesh,
        scratch_types=(
            [pltpu.VMEM((_B1, toks_per_w), jnp.int32),
             pltpu.VMEM((_NBUF, _C, N_EMBD), jnp.float32)]
            + [pltpu.SemaphoreType.DMA] * (2 * _NBUF)
        ),
    )
    def k(idxt_hbm, table_hbm, out_hbm, idx_v, rows_v, *sems):
        gsem, wsem = sems[:_NBUF], sems[_NBUF:]
        wid = lax.axis_index("s") * info.num_cores + lax.axis_index("c")
        t_base = wid * toks_per_w
        pltpu.sync_copy(idxt_hbm.at[:, pl.ds(t_base, toks_per_w)], idx_v)

        def g_issue(j, b):
            p, s = j // nsub, j % nsub
            pltpu.async_copy(table_hbm.at[idx_v.at[p, pl.ds(s * _C, _C)]],
                             rows_v.at[b], gsem[b])

        def g_wait(j, b):
            p, s = j // nsub, j % nsub
            pltpu.make_async_copy(
                table_hbm.at[idx_v.at[p, pl.ds(s * _C, _C)]],
                rows_v.at[b], gsem[b]).wait()

        def w_issue(j, b):
            p, s = j // nsub, j % nsub
            pltpu.async_copy(rows_v.at[b],
                             out_hbm.at[p, pl.ds(t_base + s * _C, _C)],
                             wsem[b])

        def w_wait(j, b):
            p, s = j // nsub, j % nsub
            pltpu.make_async_copy(
                rows_v.at[b],
                out_hbm.at[p, pl.ds(t_base + s * _C, _C)],
                wsem[b]).wait()

        for b in range(_NBUF):
            g_issue(b, b)

        g_wait(0, 0)
        w_issue(0, 0)
        for i in range(1, _NBUF):
            g_wait(i, i)
            w_issue(i, i)
            w_wait(i - 1, i - 1)
            g_issue(i + _NBUF - 1, i - 1)

        def body(g, carry):
            for i in range(_NBUF):
                j = _NBUF * g + i
                g_wait(j, i)
                w_issue(j, i)
                bb = (i + _NBUF - 1) % _NBUF
                w_wait(j - 1, bb)
                g_issue(j + _NBUF - 1, bb)
            return carry

        lax.fori_loop(1, ngrp - 1, body, 0)

        j0 = _NBUF * (ngrp - 1)
        g_wait(j0, 0)
        w_issue(j0, 0)
        w_wait(j0 - 1, _NBUF - 1)
        g_issue(j0 + _NBUF - 1, _NBUF - 1)
        for i in range(1, _NBUF):
            g_wait(j0 + i, i)
            w_issue(j0 + i, i)

        for b in range(_NBUF):
            w_wait(j0 + b, b)

    return k


_gather = _make_gather()


def kernel(x, table):
    out = _gather(x.T.astype(jnp.int32), table)
    return out.transpose(1, 0, 2)
